# baseline (speedup 1.0000x reference)
import functools

import jax
import jax.numpy as jnp
from jax import lax
from jax.experimental import pallas as pl
from jax.experimental.pallas import tpu as pltpu
from jax.experimental.pallas import tpu_sc as plsc

B = 16384
D = 16
NW = 32
BPW = B // NW
CHUNK = 128
NCHUNK = BPW // CHUNK
NGROUP = BPW // 16
H = 8


def _body(users_h, movies_h, ut_h, mt_h, w1_h, b1_h, w2_h, b2_h, out_h,
          uidx_v, midx_v, urows_v, mrows_v, w1_v, b1_v, w2_v, b2_v,
          out_v, sem):
    wid = lax.axis_index("s") * 2 + lax.axis_index("c")

    pltpu.sync_copy(w1_h, w1_v)
    pltpu.sync_copy(b1_h, b1_v)
    pltpu.sync_copy(w2_h, w2_v)
    pltpu.sync_copy(b2_h, b2_v)

    pltpu.sync_copy(users_h.at[pl.ds(wid * NCHUNK, NCHUNK)], uidx_v)
    pltpu.sync_copy(movies_h.at[pl.ds(wid * NCHUNK, NCHUNK)], midx_v)

    copies = []
    for j in range(NCHUNK):
        copies.append(pltpu.async_copy(
            ut_h.at[uidx_v.at[j]], urows_v.at[pl.ds(j * CHUNK, CHUNK)], sem))
        copies.append(pltpu.async_copy(
            mt_h.at[midx_v.at[j]], mrows_v.at[pl.ds(j * CHUNK, CHUNK)], sem))
    for c in copies:
        c.wait()

    wu = [w1_v[j, pl.ds(0, D)] for j in range(H)]
    wm = [w1_v[j, pl.ds(D, D)] for j in range(H)]
    b1r = b1_v[pl.ds(0, 16)]
    w2r = w2_v[pl.ds(0, 16)]
    b2r = b2_v[pl.ds(0, 16)]

    def group(g, carry):
        rows = g * 16 + lax.iota(jnp.int32, 16)
        accs = [jnp.full((16,), b1r[j], jnp.float32) for j in range(H)]
        for k in range(D):
            col = jnp.full((16,), k, jnp.int32)
            uk = plsc.load_gather(urows_v, [rows, col])
            mk = plsc.load_gather(mrows_v, [rows, col])
            for j in range(H):
                accs[j] = accs[j] + uk * wu[j][k] + mk * wm[j][k]
        o = jnp.full((16,), b2r[0], jnp.float32)
        for j in range(H):
            o = o + jnp.maximum(accs[j], 0.0) * w2r[j]
        out_v[pl.ds(g * 16, 16)] = jnp.maximum(o, 0.0)
        return carry

    lax.fori_loop(0, NGROUP, group, 0)

    pltpu.sync_copy(out_v, out_h.at[pl.ds(wid * BPW, BPW)])


@jax.jit
def _run(users2, movies2, user_table, movie_table, W1, b1p, W2p, b2p):
    mesh = plsc.VectorSubcoreMesh(core_axis_name="c", subcore_axis_name="s")
    call = functools.partial(
        pl.kernel,
        mesh=mesh,
        compiler_params=pltpu.CompilerParams(
            needs_layout_passes=False, use_tc_tiling_on_sc=False),
        out_type=jax.ShapeDtypeStruct((B,), jnp.float32),
        scratch_types=[
            pltpu.VMEM((NCHUNK, CHUNK), jnp.int32),
            pltpu.VMEM((NCHUNK, CHUNK), jnp.int32),
            pltpu.VMEM((BPW, D), jnp.float32),
            pltpu.VMEM((BPW, D), jnp.float32),
            pltpu.VMEM((H, 2 * D), jnp.float32),
            pltpu.VMEM((16,), jnp.float32),
            pltpu.VMEM((16,), jnp.float32),
            pltpu.VMEM((16,), jnp.float32),
            pltpu.VMEM((BPW,), jnp.float32),
            pltpu.SemaphoreType.DMA,
        ],
    )(_body)
    return call(users2, movies2, user_table, movie_table, W1, b1p, W2p, b2p)


def _pad16(x):
    return jnp.pad(x.reshape(-1), (0, 16 - x.size))


def kernel(users, movies, user_table, movie_table, W1, b1, W2, b2):
    users2 = users.reshape(B // CHUNK, CHUNK).astype(jnp.int32)
    movies2 = movies.reshape(B // CHUNK, CHUNK).astype(jnp.int32)
    out = _run(users2, movies2, user_table, movie_table, W1,
               _pad16(b1), _pad16(W2), _pad16(b2))
    return out.reshape(B, 1)


# speedup vs baseline: 3.4218x; 3.4218x over previous
import functools

import jax
import jax.numpy as jnp
from jax import lax
from jax.experimental import pallas as pl
from jax.experimental.pallas import tpu as pltpu
from jax.experimental.pallas import tpu_sc as plsc

B = 16384
D = 16
H = 8
NW = 32
BPW = B // NW

NU = 1_000_000
NM = 100_000
UBLK = NU // 128
MBLK = NM // 128
UCUT = UBLK * 128
MCUT = MBLK * 128
UTAIL = NU - UCUT
MTAIL = NM - MCUT
UT = 2 * UBLK
MT = 2 * MBLK
UROWS = UT * 64
MROWS = MT * 64

KTILES = 32
UROUNDS = 16
MROUNDS = 2


def _relayout_body(ut3, mt3, ou, om, buf, rsem, wsem):
    wid = lax.axis_index("s") * 2 + lax.axis_index("c")

    def copy_table(src, dst, ntiles, nrounds, r, _):
        for k in range(KTILES):
            t = (wid * (nrounds * KTILES) + r * KTILES + k) % ntiles
            stripe = t // (ntiles // 2)
            b = t % (ntiles // 2)
            pltpu.async_copy(src.at[stripe, :, pl.ds(b * 128, 128)],
                             buf.at[pl.ds(k * 8, 8)], rsem)
        pltpu.make_async_copy(dst.at[pl.ds(0, KTILES * 8)], buf, rsem).wait()
        for k in range(KTILES):
            t = (wid * (nrounds * KTILES) + r * KTILES + k) % ntiles
            pltpu.async_copy(buf.at[pl.ds(k * 8, 8)],
                             dst.at[pl.ds(t * 8, 8)], wsem)
        pltpu.make_async_copy(buf, dst.at[pl.ds(0, KTILES * 8)], wsem).wait()
        return _

    lax.fori_loop(0, UROUNDS,
                  functools.partial(copy_table, ut3, ou, UT, UROUNDS), 0)
    lax.fori_loop(0, MROUNDS,
                  functools.partial(copy_table, mt3, om, MT, MROUNDS), 0)


def _relayout(ut3, mt3):
    mesh = plsc.VectorSubcoreMesh(core_axis_name="c", subcore_axis_name="s")
    call = functools.partial(
        pl.kernel,
        mesh=mesh,
        out_type=(
            jax.ShapeDtypeStruct((UT * 8, 128), jnp.float32),
            jax.ShapeDtypeStruct((MT * 8, 128), jnp.float32),
        ),
        scratch_types=[
            pltpu.VMEM((KTILES * 8, 128), jnp.float32),
            pltpu.SemaphoreType.DMA,
            pltpu.SemaphoreType.DMA,
        ],
    )(_relayout_body)
    return call(ut3, mt3)


CHUNK = 128
NCHUNK = BPW // CHUNK


def _gather_mlp_body(users_h, movies_h, ul_h, ml_h, tu_h, tm_h,
                     w1_h, b1_h, w2_h, b2_h, out_h,
                     uidx_v, midx_v, rows_u, rows_m, idx_u, idx_m,
                     col_u, col_m, tl_u, tl_m,
                     w1_v, b1_v, w2_v, b2_v, out_v, sem):
    wid = lax.axis_index("s") * 2 + lax.axis_index("c")

    pltpu.sync_copy(w1_h, w1_v)
    pltpu.sync_copy(b1_h, b1_v)
    pltpu.sync_copy(w2_h, w2_v)
    pltpu.sync_copy(b2_h, b2_v)
    pltpu.sync_copy(tu_h, tl_u)
    pltpu.sync_copy(tm_h, tl_m)
    pltpu.sync_copy(users_h.at[pl.ds(wid * NCHUNK, NCHUNK)], uidx_v)
    pltpu.sync_copy(movies_h.at[pl.ds(wid * NCHUNK, NCHUNK)], midx_v)

    iota = lax.iota(jnp.int32, 16)
    offs_u = (iota >> 3) * (UBLK * 64) + (iota & 7) * 8
    offs_m = (iota >> 3) * (MBLK * 64) + (iota & 7) * 8

    wu = [w1_v[j, pl.ds(0, D)] for j in range(H)]
    wm = [w1_v[j, pl.ds(D, D)] for j in range(H)]
    b1r = b1_v[pl.ds(0, 16)]
    w2r = w2_v[pl.ds(0, 16)]
    b2r = b2_v[pl.ds(0, 16)]

    def chunk(c, carry):
        for g in range(CHUNK // 16):
            jv_u = uidx_v[c, pl.ds(g * 16, 16)]
            jc = jnp.minimum(jv_u, UCUT - 1)
            jm = jc & 127
            base = (jc >> 7) * 64 + (jm >> 4)
            col_u[pl.ds(c * CHUNK + g * 16, 16)] = jm & 15
            for t in range(16):
                p = g * 16 + t
                idx_u[p // 8, pl.ds((p % 8) * 16, 16)] = base[t] + offs_u
            jv_m = midx_v[c, pl.ds(g * 16, 16)]
            jc = jnp.minimum(jv_m, MCUT - 1)
            jm = jc & 127
            base = (jc >> 7) * 64 + (jm >> 4)
            col_m[pl.ds(c * CHUNK + g * 16, 16)] = jm & 15
            for t in range(16):
                p = g * 16 + t
                idx_m[p // 8, pl.ds((p % 8) * 16, 16)] = base[t] + offs_m

        copies = []
        for q in range(16):
            copies.append(pltpu.async_copy(
                ul_h.at[idx_u.at[q]], rows_u.at[pl.ds(q * 128, 128)], sem))
            copies.append(pltpu.async_copy(
                ml_h.at[idx_m.at[q]], rows_m.at[pl.ds(q * 128, 128)], sem))
        for cp in copies:
            cp.wait()

        for g in range(CHUNK // 16):
            jv_u = uidx_v[c, pl.ds(g * 16, 16)]
            tmask_u = jv_u >= UCUT
            trow_u = jnp.clip(jv_u - UCUT, 0, UTAIL - 1)
            jv_m = midx_v[c, pl.ds(g * 16, 16)]
            tmask_m = jv_m >= MCUT
            trow_m = jnp.clip(jv_m - MCUT, 0, MTAIL - 1)
            cv_u = col_u[pl.ds(c * CHUNK + g * 16, 16)]
            cv_m = col_m[pl.ds(c * CHUNK + g * 16, 16)]
            rowbase = (g * 16 + iota) * 16
            accs = [jnp.full((16,), b1r[j], jnp.float32) for j in range(H)]
            for k in range(D):
                kvec = jnp.full((16,), k, jnp.int32)
                uk = plsc.load_gather(rows_u, [rowbase + k, cv_u])
                tuk = plsc.load_gather(tl_u, [trow_u, kvec])
                uk = jnp.where(tmask_u, tuk, uk)
                mk = plsc.load_gather(rows_m, [rowbase + k, cv_m])
                tmk = plsc.load_gather(tl_m, [trow_m, kvec])
                mk = jnp.where(tmask_m, tmk, mk)
                for j in range(H):
                    accs[j] = accs[j] + uk * wu[j][k] + mk * wm[j][k]
            o = jnp.full((16,), b2r[0], jnp.float32)
            for j in range(H):
                o = o + jnp.maximum(accs[j], 0.0) * w2r[j]
            out_v[pl.ds(c * CHUNK + g * 16, 16)] = jnp.maximum(o, 0.0)
        return carry

    lax.fori_loop(0, NCHUNK, chunk, 0)

    pltpu.sync_copy(out_v, out_h.at[pl.ds(wid * BPW, BPW)])


def _gather_mlp(users2, movies2, u_lin, m_lin, tail_u, tail_m,
                W1, b1p, w2p, b2p):
    mesh = plsc.VectorSubcoreMesh(core_axis_name="c", subcore_axis_name="s")
    call = functools.partial(
        pl.kernel,
        mesh=mesh,
        compiler_params=pltpu.CompilerParams(
            needs_layout_passes=False, use_tc_tiling_on_sc=False),
        out_type=jax.ShapeDtypeStruct((B,), jnp.float32),
        scratch_types=[
            pltpu.VMEM((NCHUNK, CHUNK), jnp.int32),
            pltpu.VMEM((NCHUNK, CHUNK), jnp.int32),
            pltpu.VMEM((CHUNK * 16, D), jnp.float32),
            pltpu.VMEM((CHUNK * 16, D), jnp.float32),
            pltpu.VMEM((16, 128), jnp.int32),
            pltpu.VMEM((16, 128), jnp.int32),
            pltpu.VMEM((BPW,), jnp.int32),
            pltpu.VMEM((BPW,), jnp.int32),
            pltpu.VMEM((UTAIL, D), jnp.float32),
            pltpu.VMEM((MTAIL, D), jnp.float32),
            pltpu.VMEM((H, 2 * D), jnp.float32),
            pltpu.VMEM((16,), jnp.float32),
            pltpu.VMEM((16,), jnp.float32),
            pltpu.VMEM((16,), jnp.float32),
            pltpu.VMEM((BPW,), jnp.float32),
            pltpu.SemaphoreType.DMA,
        ],
    )(_gather_mlp_body)
    return call(users2, movies2, u_lin, m_lin, tail_u, tail_m,
                W1, b1p, w2p, b2p)


def _pad16(x):
    return jnp.pad(x.reshape(-1), (0, 16 - x.size))


def kernel(users, movies, user_table, movie_table, W1, b1, W2, b2):
    users2 = users.reshape(B // CHUNK, CHUNK).astype(jnp.int32)
    movies2 = movies.reshape(B // CHUNK, CHUNK).astype(jnp.int32)
    ut3 = user_table.T.reshape(2, 8, NU)
    mt3 = movie_table.T.reshape(2, 8, NM)
    ou, om = _relayout(ut3, mt3)
    u_lin = ou.reshape(UROWS, D)
    m_lin = om.reshape(MROWS, D)
    tail_u = lax.slice(user_table, (UCUT, 0), (NU, D))
    tail_m = lax.slice(movie_table, (MCUT, 0), (NM, D))
    out = _gather_mlp(users2, movies2, u_lin, m_lin, tail_u, tail_m,
                      W1, _pad16(b1), _pad16(W2), _pad16(b2))
    return out.reshape(B, 1)


# speedup vs baseline: 3.7942x; 1.1088x over previous
import functools

import jax
import jax.numpy as jnp
from jax import lax
from jax.experimental import pallas as pl
from jax.experimental.pallas import tpu as pltpu
from jax.experimental.pallas import tpu_sc as plsc

B = 16384
D = 16
H = 8
NW = 32
BPW = B // NW

NU = 1_000_000
NM = 100_000
UBLK = NU // 128
MBLK = NM // 128
UCUT = UBLK * 128
MCUT = MBLK * 128
UTAIL = NU - UCUT
MTAIL = NM - MCUT
UROWS = UCUT
MROWS = MCUT

UK = 28
URANGES = 2 * (UBLK // UK)
URPW = -(-URANGES // NW)
MK = 11
MRANGES = 2 * (MBLK // MK)
MRPW = -(-MRANGES // NW)


def _copy_table(wid, src, dst, nblk, k, rpw, stride, bufs, rsems, wsems):
    rpb = nblk // k
    tot = 2 * rpb

    def fire_reads(n, buf, rsem):
        g = (wid * rpw + n) % tot
        stripe = g // rpb
        rb = g % rpb
        return pltpu.async_copy(
            src.at[stripe, :, pl.ds(rb * k * 128, k * 128)], buf, rsem)

    def fire_writes(n, buf, wsem):
        g = (wid * rpw + n) % tot
        stripe = g // rpb
        rb = g % rpb
        return [pltpu.async_copy(
            buf.at[s],
            dst.at[pl.ds((stripe * 8 + s) * stride + rb * k * 128, k * 128)],
            wsem) for s in range(8)]

    robj = {0: fire_reads(0, bufs[0], rsems[0])}
    if rpw > 1:
        robj[1] = fire_reads(1, bufs[1], rsems[1])
    wobj = {}
    for n in range(rpw):
        if n + 2 < rpw:
            if n - 1 >= 0:
                for o in wobj[n - 1]:
                    o.wait()
            robj[n + 2] = fire_reads(n + 2, bufs[(n + 2) % 3],
                                     rsems[(n + 2) % 3])
        robj[n].wait()
        wobj[n] = fire_writes(n, bufs[n % 3], wsems[n % 3])
    for n in (rpw - 2, rpw - 1):
        if n >= 0:
            for o in wobj[n]:
                o.wait()


def _relayout_body(ut3, mt3, ou, om,
                   ub0, ub1, ub2, mb0, mb1, mb2,
                   rs0, rs1, rs2, ws0, ws1, ws2):
    wid = lax.axis_index("s") * 2 + lax.axis_index("c")
    _copy_table(wid, ut3, ou, UBLK, UK, URPW, UCUT,
                (ub0, ub1, ub2), (rs0, rs1, rs2), (ws0, ws1, ws2))
    _copy_table(wid, mt3, om, MBLK, MK, MRPW, MCUT,
                (mb0, mb1, mb2), (rs0, rs1, rs2), (ws0, ws1, ws2))


def _relayout(ut3, mt3):
    mesh = plsc.VectorSubcoreMesh(core_axis_name="c", subcore_axis_name="s")
    call = functools.partial(
        pl.kernel,
        mesh=mesh,
        out_type=(
            jax.ShapeDtypeStruct((16 * UCUT,), jnp.float32),
            jax.ShapeDtypeStruct((16 * MCUT,), jnp.float32),
        ),
        scratch_types=(
            [pltpu.VMEM((8, UK * 128), jnp.float32) for _ in range(3)]
            + [pltpu.VMEM((8, MK * 128), jnp.float32) for _ in range(3)]
            + [pltpu.SemaphoreType.DMA] * 6
        ),
    )(_relayout_body)
    return call(ut3, mt3)


CHUNK = 128
NCHUNK = BPW // CHUNK


def _gather_mlp_body(users_h, movies_h, ul_h, ml_h, tu_h, tm_h,
                     w1_h, b1_h, w2_h, b2_h, out_h,
                     uidx_v, midx_v, rows_u, rows_m, idx_u, idx_m,
                     col_u, col_m, tl_u, tl_m,
                     w1_v, b1_v, w2_v, b2_v, out_v, sem):
    wid = lax.axis_index("s") * 2 + lax.axis_index("c")

    pltpu.sync_copy(w1_h, w1_v)
    pltpu.sync_copy(b1_h, b1_v)
    pltpu.sync_copy(w2_h, w2_v)
    pltpu.sync_copy(b2_h, b2_v)
    pltpu.sync_copy(tu_h, tl_u)
    pltpu.sync_copy(tm_h, tl_m)
    pltpu.sync_copy(users_h.at[pl.ds(wid * NCHUNK, NCHUNK)], uidx_v)
    pltpu.sync_copy(movies_h.at[pl.ds(wid * NCHUNK, NCHUNK)], midx_v)

    iota = lax.iota(jnp.int32, 16)
    offs_u = iota * (UROWS // 16)
    offs_m = iota * (MROWS // 16)

    wu = [w1_v[j, pl.ds(0, D)] for j in range(H)]
    wm = [w1_v[j, pl.ds(D, D)] for j in range(H)]
    b1r = b1_v[pl.ds(0, 16)]
    w2r = w2_v[pl.ds(0, 16)]
    b2r = b2_v[pl.ds(0, 16)]

    def chunk(c, carry):
        for g in range(CHUNK // 16):
            jv_u = uidx_v[c, pl.ds(g * 16, 16)]
            jc = jnp.minimum(jv_u, UCUT - 1)
            base = jc >> 4
            col_u[pl.ds(c * CHUNK + g * 16, 16)] = jc & 15
            for t in range(16):
                p = g * 16 + t
                idx_u[p // 8, pl.ds((p % 8) * 16, 16)] = base[t] + offs_u
            jv_m = midx_v[c, pl.ds(g * 16, 16)]
            jc = jnp.minimum(jv_m, MCUT - 1)
            base = jc >> 4
            col_m[pl.ds(c * CHUNK + g * 16, 16)] = jc & 15
            for t in range(16):
                p = g * 16 + t
                idx_m[p // 8, pl.ds((p % 8) * 16, 16)] = base[t] + offs_m

        copies = []
        for q in range(16):
            copies.append(pltpu.async_copy(
                ul_h.at[idx_u.at[q]], rows_u.at[pl.ds(q * 128, 128)], sem))
            copies.append(pltpu.async_copy(
                ml_h.at[idx_m.at[q]], rows_m.at[pl.ds(q * 128, 128)], sem))
        for cp in copies:
            cp.wait()

        for g in range(CHUNK // 16):
            jv_u = uidx_v[c, pl.ds(g * 16, 16)]
            tmask_u = jv_u >= UCUT
            trow_u = jnp.clip(jv_u - UCUT, 0, UTAIL - 1)
            jv_m = midx_v[c, pl.ds(g * 16, 16)]
            tmask_m = jv_m >= MCUT
            trow_m = jnp.clip(jv_m - MCUT, 0, MTAIL - 1)
            cv_u = col_u[pl.ds(c * CHUNK + g * 16, 16)]
            cv_m = col_m[pl.ds(c * CHUNK + g * 16, 16)]
            rowbase = (g * 16 + iota) * 16
            accs = [jnp.full((16,), b1r[j], jnp.float32) for j in range(H)]
            for k in range(D):
                kvec = jnp.full((16,), k, jnp.int32)
                uk = plsc.load_gather(rows_u, [rowbase + k, cv_u])
                tuk = plsc.load_gather(tl_u, [trow_u, kvec])
                uk = jnp.where(tmask_u, tuk, uk)
                mk = plsc.load_gather(rows_m, [rowbase + k, cv_m])
                tmk = plsc.load_gather(tl_m, [trow_m, kvec])
                mk = jnp.where(tmask_m, tmk, mk)
                for j in range(H):
                    accs[j] = accs[j] + uk * wu[j][k] + mk * wm[j][k]
            o = jnp.full((16,), b2r[0], jnp.float32)
            for j in range(H):
                o = o + jnp.maximum(accs[j], 0.0) * w2r[j]
            out_v[pl.ds(c * CHUNK + g * 16, 16)] = jnp.maximum(o, 0.0)
        return carry

    lax.fori_loop(0, NCHUNK, chunk, 0)

    pltpu.sync_copy(out_v, out_h.at[pl.ds(wid * BPW, BPW)])


def _gather_mlp(users2, movies2, u_lin, m_lin, tail_u, tail_m,
                W1, b1p, w2p, b2p):
    mesh = plsc.VectorSubcoreMesh(core_axis_name="c", subcore_axis_name="s")
    call = functools.partial(
        pl.kernel,
        mesh=mesh,
        compiler_params=pltpu.CompilerParams(
            needs_layout_passes=False, use_tc_tiling_on_sc=False),
        out_type=jax.ShapeDtypeStruct((B,), jnp.float32),
        scratch_types=[
            pltpu.VMEM((NCHUNK, CHUNK), jnp.int32),
            pltpu.VMEM((NCHUNK, CHUNK), jnp.int32),
            pltpu.VMEM((CHUNK * 16, D), jnp.float32),
            pltpu.VMEM((CHUNK * 16, D), jnp.float32),
            pltpu.VMEM((16, 128), jnp.int32),
            pltpu.VMEM((16, 128), jnp.int32),
            pltpu.VMEM((BPW,), jnp.int32),
            pltpu.VMEM((BPW,), jnp.int32),
            pltpu.VMEM((UTAIL, D), jnp.float32),
            pltpu.VMEM((MTAIL, D), jnp.float32),
            pltpu.VMEM((H, 2 * D), jnp.float32),
            pltpu.VMEM((16,), jnp.float32),
            pltpu.VMEM((16,), jnp.float32),
            pltpu.VMEM((16,), jnp.float32),
            pltpu.VMEM((BPW,), jnp.float32),
            pltpu.SemaphoreType.DMA,
        ],
    )(_gather_mlp_body)
    return call(users2, movies2, u_lin, m_lin, tail_u, tail_m,
                W1, b1p, w2p, b2p)


def _pad16(x):
    return jnp.pad(x.reshape(-1), (0, 16 - x.size))


def kernel(users, movies, user_table, movie_table, W1, b1, W2, b2):
    users2 = users.reshape(B // CHUNK, CHUNK).astype(jnp.int32)
    movies2 = movies.reshape(B // CHUNK, CHUNK).astype(jnp.int32)
    ut3 = user_table.T.reshape(2, 8, NU)
    mt3 = movie_table.T.reshape(2, 8, NM)
    ou, om = _relayout(ut3, mt3)
    u_lin = ou.reshape(UROWS, D)
    m_lin = om.reshape(MROWS, D)
    tail_u = lax.slice(user_table, (UCUT, 0), (NU, D))
    tail_m = lax.slice(movie_table, (MCUT, 0), (NM, D))
    out = _gather_mlp(users2, movies2, u_lin, m_lin, tail_u, tail_m,
                      W1, _pad16(b1), _pad16(W2), _pad16(b2))
    return out.reshape(B, 1)


# speedup vs baseline: 4.0540x; 1.0685x over previous
import functools

import jax
import jax.numpy as jnp
from jax import lax
from jax.experimental import pallas as pl
from jax.experimental.pallas import tpu as pltpu
from jax.experimental.pallas import tpu_sc as plsc

B = 16384
D = 16
H = 8
NW = 32
BPW = B // NW

NU = 1_000_000
NM = 100_000
UBLK = NU // 128
MBLK = NM // 128
UCUT = UBLK * 128
MCUT = MBLK * 128
UTAIL = NU - UCUT
MTAIL = NM - MCUT
UROWS = UCUT
MROWS = MCUT

UK = 28
URANGES = 2 * (UBLK // UK)
URPW = -(-URANGES // NW)
MK = 11
MRANGES = 2 * (MBLK // MK)
MRPW = -(-MRANGES // NW)


def _copy_table(wid, src, dst, nblk, k, rpw, stride, bufs, rsems, wsems):
    rpb = nblk // k
    tot = 2 * rpb

    def fire_reads(n, buf, rsem):
        g = (wid * rpw + n) % tot
        stripe = g // rpb
        rb = g % rpb
        return pltpu.async_copy(
            src.at[stripe, :, pl.ds(rb * k * 128, k * 128)], buf, rsem)

    def fire_writes(n, buf, wsem):
        g = (wid * rpw + n) % tot
        stripe = g // rpb
        rb = g % rpb
        return [pltpu.async_copy(
            buf.at[s],
            dst.at[pl.ds((stripe * 8 + s) * stride + rb * k * 128, k * 128)],
            wsem) for s in range(8)]

    robj = {0: fire_reads(0, bufs[0], rsems[0])}
    if rpw > 1:
        robj[1] = fire_reads(1, bufs[1], rsems[1])
    wobj = {}
    for n in range(rpw):
        if n + 2 < rpw:
            if n - 1 >= 0:
                for o in wobj[n - 1]:
                    o.wait()
            robj[n + 2] = fire_reads(n + 2, bufs[(n + 2) % 3],
                                     rsems[(n + 2) % 3])
        robj[n].wait()
        wobj[n] = fire_writes(n, bufs[n % 3], wsems[n % 3])
    for n in (rpw - 2, rpw - 1):
        if n >= 0:
            for o in wobj[n]:
                o.wait()


def _relayout_body(ut3, mt3, ou, om,
                   ub0, ub1, ub2, mb0, mb1, mb2,
                   rs0, rs1, rs2, ws0, ws1, ws2):
    wid = lax.axis_index("s") * 2 + lax.axis_index("c")
    _copy_table(wid, ut3, ou, UBLK, UK, URPW, UCUT,
                (ub0, ub1, ub2), (rs0, rs1, rs2), (ws0, ws1, ws2))
    _copy_table(wid, mt3, om, MBLK, MK, MRPW, MCUT,
                (mb0, mb1, mb2), (rs0, rs1, rs2), (ws0, ws1, ws2))


def _relayout(ut3, mt3):
    mesh = plsc.VectorSubcoreMesh(core_axis_name="c", subcore_axis_name="s")
    call = functools.partial(
        pl.kernel,
        mesh=mesh,
        out_type=(
            jax.ShapeDtypeStruct((16 * UCUT,), jnp.float32),
            jax.ShapeDtypeStruct((16 * MCUT,), jnp.float32),
        ),
        scratch_types=(
            [pltpu.VMEM((8, UK * 128), jnp.float32) for _ in range(3)]
            + [pltpu.VMEM((8, MK * 128), jnp.float32) for _ in range(3)]
            + [pltpu.SemaphoreType.DMA] * 6
        ),
    )(_relayout_body)
    return call(ut3, mt3)


CHUNK = 64
NCHUNK = BPW // CHUNK
NG = CHUNK // 16
NSTR = CHUNK * 16 // 128


def _gather_mlp_body(users_h, movies_h, ul_h, ml_h, tu_h, tm_h,
                     w1_h, b1_h, w2_h, b2_h, out_h,
                     uidx_v, midx_v, ru_a, ru_b, rm_a, rm_b,
                     iu_a, iu_b, im_a, im_b,
                     col_u, col_m, tl_u, tl_m,
                     w1_v, b1_v, w2_v, b2_v, out_v, sem_a, sem_b):
    wid = lax.axis_index("s") * 2 + lax.axis_index("c")

    pltpu.sync_copy(w1_h, w1_v)
    pltpu.sync_copy(b1_h, b1_v)
    pltpu.sync_copy(w2_h, w2_v)
    pltpu.sync_copy(b2_h, b2_v)
    pltpu.sync_copy(tu_h, tl_u)
    pltpu.sync_copy(tm_h, tl_m)
    pltpu.sync_copy(users_h.at[pl.ds(wid * 4, 4)], uidx_v)
    pltpu.sync_copy(movies_h.at[pl.ds(wid * 4, 4)], midx_v)

    iota = lax.iota(jnp.int32, 16)
    offs_u = iota * (UROWS // 16)
    offs_m = iota * (MROWS // 16)

    wu = [w1_v[j, pl.ds(0, D)] for j in range(H)]
    wm = [w1_v[j, pl.ds(D, D)] for j in range(H)]
    b1r = b1_v[pl.ds(0, 16)]
    w2r = w2_v[pl.ds(0, 16)]
    b2r = b2_v[pl.ds(0, 16)]

    def build_fire(c, ru, rm, iu, im, sem):
        for g in range(NG):
            jv = uidx_v[c >> 1, pl.ds((c & 1) * CHUNK + g * 16, 16)]
            jc = jnp.minimum(jv, UCUT - 1)
            base = jc >> 4
            col_u[pl.ds(c * CHUNK + g * 16, 16)] = jc & 15
            for t in range(16):
                p = g * 16 + t
                iu[p // 8, pl.ds((p % 8) * 16, 16)] = base[t] + offs_u
            jv = midx_v[c >> 1, pl.ds((c & 1) * CHUNK + g * 16, 16)]
            jc = jnp.minimum(jv, MCUT - 1)
            base = jc >> 4
            col_m[pl.ds(c * CHUNK + g * 16, 16)] = jc & 15
            for t in range(16):
                p = g * 16 + t
                im[p // 8, pl.ds((p % 8) * 16, 16)] = base[t] + offs_m
        for q in range(NSTR):
            pltpu.async_copy(ul_h.at[iu.at[q]],
                             ru.at[pl.ds(q * 128, 128)], sem)
            pltpu.async_copy(ml_h.at[im.at[q]],
                             rm.at[pl.ds(q * 128, 128)], sem)

    def drain(ru, rm, sem):
        pltpu.make_async_copy(ul_h.at[pl.ds(0, CHUNK * 16)], ru, sem).wait()
        pltpu.make_async_copy(ml_h.at[pl.ds(0, CHUNK * 16)], rm, sem).wait()

    def mlp(c, ru, rm):
        for g in range(NG):
            jv_u = uidx_v[c >> 1, pl.ds((c & 1) * CHUNK + g * 16, 16)]
            tmask_u = jv_u >= UCUT
            trow_u = jnp.clip(jv_u - UCUT, 0, UTAIL - 1)
            jv_m = midx_v[c >> 1, pl.ds((c & 1) * CHUNK + g * 16, 16)]
            tmask_m = jv_m >= MCUT
            trow_m = jnp.clip(jv_m - MCUT, 0, MTAIL - 1)
            cv_u = col_u[pl.ds(c * CHUNK + g * 16, 16)]
            cv_m = col_m[pl.ds(c * CHUNK + g * 16, 16)]
            rowbase = (g * 16 + iota) * 16
            accs = [jnp.full((16,), b1r[j], jnp.float32) for j in range(H)]
            for k in range(D):
                kvec = jnp.full((16,), k, jnp.int32)
                uk = plsc.load_gather(ru, [rowbase + k, cv_u])
                tuk = plsc.load_gather(tl_u, [trow_u, kvec])
                uk = jnp.where(tmask_u, tuk, uk)
                mk = plsc.load_gather(rm, [rowbase + k, cv_m])
                tmk = plsc.load_gather(tl_m, [trow_m, kvec])
                mk = jnp.where(tmask_m, tmk, mk)
                for j in range(H):
                    accs[j] = accs[j] + uk * wu[j][k] + mk * wm[j][k]
            o = jnp.full((16,), b2r[0], jnp.float32)
            for j in range(H):
                o = o + jnp.maximum(accs[j], 0.0) * w2r[j]
            out_v[pl.ds(c * CHUNK + g * 16, 16)] = jnp.maximum(o, 0.0)

    build_fire(0, ru_a, rm_a, iu_a, im_a, sem_a)

    def pair(i, carry):
        c0 = 2 * i
        c1 = 2 * i + 1
        build_fire(c1, ru_b, rm_b, iu_b, im_b, sem_b)
        drain(ru_a, rm_a, sem_a)
        mlp(c0, ru_a, rm_a)
        build_fire(jnp.minimum(c1 + 1, NCHUNK - 1), ru_a, rm_a, iu_a, im_a,
                   sem_a)
        drain(ru_b, rm_b, sem_b)
        mlp(c1, ru_b, rm_b)
        return carry

    lax.fori_loop(0, NCHUNK // 2, pair, 0)
    drain(ru_a, rm_a, sem_a)

    pltpu.sync_copy(out_v, out_h.at[pl.ds(wid * BPW, BPW)])


def _gather_mlp(users2, movies2, u_lin, m_lin, tail_u, tail_m,
                W1, b1p, w2p, b2p):
    mesh = plsc.VectorSubcoreMesh(core_axis_name="c", subcore_axis_name="s")
    call = functools.partial(
        pl.kernel,
        mesh=mesh,
        compiler_params=pltpu.CompilerParams(
            needs_layout_passes=False, use_tc_tiling_on_sc=False),
        out_type=jax.ShapeDtypeStruct((B,), jnp.float32),
        scratch_types=[
            pltpu.VMEM((4, 128), jnp.int32),
            pltpu.VMEM((4, 128), jnp.int32),
            pltpu.VMEM((CHUNK * 16, D), jnp.float32),
            pltpu.VMEM((CHUNK * 16, D), jnp.float32),
            pltpu.VMEM((CHUNK * 16, D), jnp.float32),
            pltpu.VMEM((CHUNK * 16, D), jnp.float32),
            pltpu.VMEM((NSTR, 128), jnp.int32),
            pltpu.VMEM((NSTR, 128), jnp.int32),
            pltpu.VMEM((NSTR, 128), jnp.int32),
            pltpu.VMEM((NSTR, 128), jnp.int32),
            pltpu.VMEM((BPW,), jnp.int32),
            pltpu.VMEM((BPW,), jnp.int32),
            pltpu.VMEM((UTAIL, D), jnp.float32),
            pltpu.VMEM((MTAIL, D), jnp.float32),
            pltpu.VMEM((H, 2 * D), jnp.float32),
            pltpu.VMEM((16,), jnp.float32),
            pltpu.VMEM((16,), jnp.float32),
            pltpu.VMEM((16,), jnp.float32),
            pltpu.VMEM((BPW,), jnp.float32),
            pltpu.SemaphoreType.DMA,
            pltpu.SemaphoreType.DMA,
        ],
    )(_gather_mlp_body)
    return call(users2, movies2, u_lin, m_lin, tail_u, tail_m,
                W1, b1p, w2p, b2p)


def _pad16(x):
    return jnp.pad(x.reshape(-1), (0, 16 - x.size))


def kernel(users, movies, user_table, movie_table, W1, b1, W2, b2):
    users2 = users.reshape(B // 128, 128).astype(jnp.int32)
    movies2 = movies.reshape(B // 128, 128).astype(jnp.int32)
    ut3 = user_table.T.reshape(2, 8, NU)
    mt3 = movie_table.T.reshape(2, 8, NM)
    ou, om = _relayout(ut3, mt3)
    u_lin = ou.reshape(UROWS, D)
    m_lin = om.reshape(MROWS, D)
    tail_u = lax.slice(user_table, (UCUT, 0), (NU, D))
    tail_m = lax.slice(movie_table, (MCUT, 0), (NM, D))
    out = _gather_mlp(users2, movies2, u_lin, m_lin, tail_u, tail_m,
                      W1, _pad16(b1), _pad16(W2), _pad16(b2))
    return out.reshape(B, 1)
